# baseline (speedup 1.0000x reference)
import functools

import numpy as np
import jax
import jax.numpy as jnp
from jax import lax
from jax.experimental import pallas as pl
from jax.experimental.pallas import tpu as pltpu
from jax.experimental.pallas import tpu_sc as plsc

_MAX_SEQ_LEN = 200
_EMBED_DIM = 64


def _pos_table(max_seq_length, embed_dim):
    pe = np.zeros((max_seq_length, embed_dim), dtype=np.float64)
    pos = np.arange(max_seq_length, dtype=np.float64)[:, None]
    i_even = np.arange(0, embed_dim, 2, dtype=np.float64)
    pe[:, 0::2] = np.sin(pos / np.power(10000.0, i_even / embed_dim))
    pe[:, 1::2] = np.cos(pos / np.power(10000.0, (i_even + 1.0) / embed_dim))
    return pe.astype(np.float32)


_POS_NP = _pos_table(_MAX_SEQ_LEN, _EMBED_DIM)

_NC = 2
_NS = 16
_NW = _NC * _NS
_LANES = 16


@functools.partial(jax.jit, static_argnames=("batch", "seq"))
def _embed_lookup(idx_flat, table, pos, *, batch, seq):
    d = table.shape[1]
    tot = batch * seq
    rows_per_w = tot // _NW
    ch = seq
    n_chunks = rows_per_w // ch

    mesh = plsc.VectorSubcoreMesh(core_axis_name="c", subcore_axis_name="s")

    @functools.partial(
        pl.kernel,
        mesh=mesh,
        out_type=jax.ShapeDtypeStruct((tot, d), jnp.float32),
        scratch_types=[
            pltpu.VMEM((ch,), jnp.int32),
            pltpu.VMEM((ch, d), jnp.float32),
            pltpu.VMEM((seq, d), jnp.float32),
            pltpu.SemaphoreType.DMA,
        ],
        compiler_params=pltpu.CompilerParams(use_tc_tiling_on_sc=False),
    )
    def _k(idx_hbm, table_hbm, pos_hbm, out_hbm, idx_v, rows_v, pos_v, sem):
        wid = lax.axis_index("s") * _NC + lax.axis_index("c")
        base = wid * rows_per_w
        pltpu.sync_copy(pos_hbm, pos_v)

        def chunk_body(c, _):
            off = base + c * ch
            pltpu.sync_copy(idx_hbm.at[pl.ds(off, ch)], idx_v)
            pltpu.async_copy(table_hbm.at[idx_v], rows_v, sem).wait()

            def row_body(r, _):
                for j in range(d // _LANES):
                    sl = pl.ds(j * _LANES, _LANES)
                    plsc.addupdate(rows_v.at[r, sl], pos_v[r, sl])
                return 0

            lax.fori_loop(0, ch, row_body, 0, unroll=2)
            pltpu.sync_copy(rows_v, out_hbm.at[pl.ds(off, ch)])
            return 0

        lax.fori_loop(0, n_chunks, chunk_body, 0)

    return _k(idx_flat, table, pos)


def kernel(inputs, token_table):
    batch, seq = inputs.shape
    idx_flat = inputs.astype(jnp.int32).reshape(-1)
    pos = jnp.asarray(_POS_NP[:seq])
    out = _embed_lookup(idx_flat, token_table, pos, batch=batch, seq=seq)
    return out.reshape(batch, seq, token_table.shape[1])
